# baseline (speedup 1.0000x reference)
import functools

import jax
import jax.numpy as jnp
from jax import lax
from jax.experimental import pallas as pl
from jax.experimental.pallas import tpu as pltpu
from jax.experimental.pallas import tpu_sc as plsc

B = 4
S = 2048
D = 768

_info = plsc.get_sparse_core_info()
_NC = _info.num_cores
_NS = _info.num_subcores
_L = _info.num_lanes
_NW = _NC * _NS
_S_PER_W = S // _NW
_VECS = D // _L

_mesh = plsc.VectorSubcoreMesh(core_axis_name="c", subcore_axis_name="s")


@functools.partial(
    pl.kernel,
    mesh=_mesh,
    out_type=jax.ShapeDtypeStruct((B, S, D), jnp.float32),
    scratch_types=[
        pltpu.VMEM((_S_PER_W,), jnp.int32),
        pltpu.VMEM((_S_PER_W, D), jnp.float32),
        pltpu.VMEM((_S_PER_W, D), jnp.float32),
        pltpu.SemaphoreType.DMA,
    ],
)
def _embed(ids_hbm, word_hbm, pos_hbm, out_hbm, idx_v, pos_v, w_v, sem):
    wid = lax.axis_index("s") * _NC + lax.axis_index("c")
    s_base = wid * _S_PER_W

    pltpu.sync_copy(pos_hbm.at[pl.ds(s_base, _S_PER_W)], pos_v)

    for b in range(B):
        pltpu.sync_copy(ids_hbm.at[b, pl.ds(s_base, _S_PER_W)], idx_v)
        pltpu.async_copy(word_hbm.at[idx_v], w_v, sem).wait()

        def _row(r, carry):
            for c in range(_VECS):
                sl = pl.ds(c * _L, _L)
                w_v[r, sl] = w_v[r, sl] + pos_v[r, sl]
            return carry

        lax.fori_loop(0, _S_PER_W, _row, 0)
        pltpu.sync_copy(w_v, out_hbm.at[b, pl.ds(s_base, _S_PER_W)])


def kernel(ids, word_table, pos_table):
    return _embed(ids.astype(jnp.int32), word_table, pos_table)
